# baseline (speedup 1.0000x reference)
import functools

import jax
import jax.numpy as jnp
from jax import lax
from jax.experimental import pallas as pl
from jax.experimental.pallas import tpu as pltpu
from jax.experimental.pallas import tpu_sc as plsc

N = 10000
E = 320000
D = 128
H1 = 32
H2 = 16
OUT = 2

NC = 2
NS = 16
NW = NC * NS
CHUNK = 128
NROWS = 2560
E_PAD = NROWS * CHUNK
R = NROWS // NW
N_PAD = 10112
RPT = N_PAD // NS
TB = 2000
CW = 8
NBUF = 4


def _seg_sum_kernel(F, with_cnt):
    mesh = plsc.VectorSubcoreMesh(core_axis_name="c", subcore_axis_name="s")
    out_type = [jax.ShapeDtypeStruct((NC, N_PAD, F), jnp.float32)]
    scratch = [
        pltpu.VMEM((R, CHUNK), jnp.int32),
        pltpu.VMEM((R, CHUNK), jnp.int32),
        pltpu.VMEM((NBUF, CHUNK, F), jnp.float32),
        pltpu.VMEM_SHARED((N_PAD, F), jnp.float32),
        pltpu.VMEM_SHARED((N_PAD, F), jnp.float32),
        pltpu.SemaphoreType.DMA((NBUF,)),
        pltpu.SemaphoreType.DMA((NBUF,)),
        pltpu.SemaphoreType.DMA,
    ]
    if with_cnt:
        out_type.append(jax.ShapeDtypeStruct((NC, N_PAD, CW), jnp.float32))
        scratch += [
            pltpu.VMEM((CHUNK, CW), jnp.float32),
            pltpu.VMEM_SHARED((N_PAD, CW), jnp.float32),
        ]

    def body(src_hbm, dst_hbm, y_hbm, zacc_hbm, ones_hbm, zcnt_hbm,
             *rest):
        if with_cnt:
            (out_hbm, cnt_hbm,
             src_v, dst_v, rows_v, y_sh, acc_sh, gsems, ssems, csem,
             ones_v, cnt_sh) = rest
        else:
            (out_hbm, src_v, dst_v, rows_v, y_sh, acc_sh, gsems, ssems,
             csem) = rest
        c = lax.axis_index("c")
        s = lax.axis_index("s")
        wid = s * NC + c
        pltpu.sync_copy(y_hbm.at[pl.ds(s * RPT, RPT)],
                        y_sh.at[pl.ds(s * RPT, RPT)])
        pltpu.sync_copy(zacc_hbm, acc_sh.at[pl.ds(s * RPT, RPT)])
        if with_cnt:
            pltpu.sync_copy(zcnt_hbm, cnt_sh.at[pl.ds(s * RPT, RPT)])
            pltpu.sync_copy(ones_hbm, ones_v)
        pltpu.sync_copy(src_hbm.at[pl.ds(wid * R, R)], src_v)
        pltpu.sync_copy(dst_hbm.at[pl.ds(wid * R, R)], dst_v)
        plsc.subcore_barrier()

        for b in range(NBUF):
            pltpu.async_copy(y_sh.at[src_v.at[b]], rows_v.at[b], gsems.at[b])

        def group(g, carry):
            j0 = g * NBUF
            for b in range(NBUF):
                j = j0 + b
                pltpu.make_async_copy(y_sh.at[src_v.at[j]], rows_v.at[b],
                                      gsems.at[b]).wait()
                pltpu.async_copy(rows_v.at[b], acc_sh.at[dst_v.at[j]],
                                 ssems.at[b], add=True)
                if with_cnt:
                    pltpu.async_copy(ones_v, cnt_sh.at[dst_v.at[j]], csem,
                                     add=True)
            for b in range(NBUF):
                j = j0 + b
                pltpu.make_async_copy(rows_v.at[b], acc_sh.at[dst_v.at[j]],
                                      ssems.at[b]).wait()
                jn = lax.rem(j + NBUF, R)
                pltpu.async_copy(y_sh.at[src_v.at[jn]], rows_v.at[b],
                                 gsems.at[b])
            return carry

        lax.fori_loop(0, R // NBUF, group, 0)
        for b in range(NBUF):
            pltpu.make_async_copy(y_sh.at[src_v.at[b]], rows_v.at[b],
                                  gsems.at[b]).wait()
        if with_cnt:
            def drain_cnt(j, carry):
                pltpu.make_async_copy(ones_v, cnt_sh.at[dst_v.at[0]],
                                      csem).wait()
                return carry
            lax.fori_loop(0, R, drain_cnt, 0)
        plsc.subcore_barrier()
        pltpu.sync_copy(acc_sh.at[pl.ds(s * RPT, RPT)],
                        out_hbm.at[c, pl.ds(s * RPT, RPT)])
        if with_cnt:
            pltpu.sync_copy(cnt_sh.at[pl.ds(s * RPT, RPT)],
                            cnt_hbm.at[c, pl.ds(s * RPT, RPT)])

    return pl.kernel(body, out_type=out_type if with_cnt else out_type[0],
                     mesh=mesh, scratch_types=scratch,
                     compiler_params=pltpu.CompilerParams(
                         use_tc_tiling_on_sc=False))


_DN = (((1,), (1,)), ((), ()))


def _t1_body(x_ref, wl_ref, wr_ref, b_ref, y_ref, r_ref):
    x = x_ref[...]
    y_ref[...] = lax.dot_general(x, wl_ref[...], _DN,
                                 preferred_element_type=jnp.float32)
    r_ref[...] = lax.dot_general(x, wr_ref[...], _DN,
                                 preferred_element_type=jnp.float32) + b_ref[...]


def _t2_body(acc_ref, cnt_ref, r1_ref, wl_ref, wr_ref, b_ref,
             y2_ref, r2_ref, inv_ref):
    agg = acc_ref[0] + acc_ref[1]
    cnt = cnt_ref[0, :, 0:1] + cnt_ref[1, :, 0:1]
    inv = 1.0 / jnp.maximum(cnt, 1.0)
    h1 = jnp.maximum(agg * inv + r1_ref[...], 0.0)
    y2_ref[...] = lax.dot_general(h1, wl_ref[...], _DN,
                                  preferred_element_type=jnp.float32)
    r2_ref[...] = lax.dot_general(h1, wr_ref[...], _DN,
                                  preferred_element_type=jnp.float32) + b_ref[...]
    inv_ref[...] = jnp.broadcast_to(inv, (TB, H2))


def _t3_body(acc_ref, r2_ref, inv_ref, wlin_ref, blin_ref, out_ref):
    agg = acc_ref[0] + acc_ref[1]
    h2 = jnp.maximum(agg * inv_ref[...] + r2_ref[...], 0.0)
    out_ref[...] = lax.dot_general(h2, wlin_ref[...], _DN,
                                   preferred_element_type=jnp.float32) + blin_ref[...]


def _row_blocked(shape3=False, width=None):
    if shape3:
        return pl.BlockSpec((NC, TB, width), lambda i: (0, i, 0))
    return pl.BlockSpec((TB, width), lambda i: (i, 0))


def _full(shape):
    return pl.BlockSpec(shape, lambda i: tuple(0 for _ in shape))


_t1 = pl.pallas_call(
    _t1_body,
    grid=(N // TB,),
    in_specs=[_row_blocked(width=D), _full((H1, D)), _full((H1, D)),
              _full((1, H1))],
    out_specs=[_row_blocked(width=H1), _row_blocked(width=H1)],
    out_shape=[
        jax.ShapeDtypeStruct((N_PAD, H1), jnp.float32),
        jax.ShapeDtypeStruct((N_PAD, H1), jnp.float32),
    ])
_t2 = pl.pallas_call(
    _t2_body,
    grid=(N // TB,),
    in_specs=[_row_blocked(True, H1), _row_blocked(True, CW),
              _row_blocked(width=H1), _full((H2, H1)), _full((H2, H1)),
              _full((1, H2))],
    out_specs=[_row_blocked(width=H2), _row_blocked(width=H2),
               _row_blocked(width=H2)],
    out_shape=[
        jax.ShapeDtypeStruct((N_PAD, H2), jnp.float32),
        jax.ShapeDtypeStruct((N_PAD, H2), jnp.float32),
        jax.ShapeDtypeStruct((N_PAD, H2), jnp.float32),
    ])
_t3 = pl.pallas_call(
    _t3_body,
    grid=(N // TB,),
    in_specs=[_row_blocked(True, H2), _row_blocked(width=H2),
              _row_blocked(width=H2), _full((OUT, H2)), _full((1, OUT))],
    out_specs=_row_blocked(width=OUT),
    out_shape=jax.ShapeDtypeStruct((N, OUT), jnp.float32))

_s1 = _seg_sum_kernel(H1, with_cnt=True)
_s2 = _seg_sum_kernel(H2, with_cnt=False)


def kernel(x, edge_index, W1l, b1, W1r, W2l, b2, W2r, Wlin, blin):
    pad = E_PAD - E
    trash = N + (jnp.arange(pad, dtype=jnp.int32) % (N_PAD - N))
    src = jnp.concatenate([edge_index[0], jnp.zeros((pad,), jnp.int32)])
    dst = jnp.concatenate([edge_index[1], trash])
    src = src.reshape(NROWS, CHUNK)
    dst = dst.reshape(NROWS, CHUNK)
    zacc1 = jnp.zeros((RPT, H1), jnp.float32)
    zacc2 = jnp.zeros((RPT, H2), jnp.float32)
    zcnt = jnp.zeros((RPT, CW), jnp.float32)
    ones = jnp.ones((CHUNK, CW), jnp.float32)

    y1, r1 = _t1(x, W1l, W1r, b1.reshape(1, H1))
    acc1, cntp = _s1(src, dst, y1, zacc1, ones, zcnt)
    y2, r2, inv = _t2(acc1, cntp, r1, W2l, W2r, b2.reshape(1, H2))
    acc2 = _s2(src, dst, y2, zacc2, ones, zcnt)
    out = _t3(acc2, r2, inv, Wlin, blin.reshape(1, OUT))
    return out


# speedup vs baseline: 1.0932x; 1.0932x over previous
import functools

import jax
import jax.numpy as jnp
from jax import lax
from jax.experimental import pallas as pl
from jax.experimental.pallas import tpu as pltpu
from jax.experimental.pallas import tpu_sc as plsc

N = 10000
E = 320000
D = 128
H1 = 32
H2 = 16
OUT = 2

NC = 2
NS = 16
NW = NC * NS
CHUNK = 128
NROWS = 2560
E_PAD = NROWS * CHUNK
R = NROWS // NW
N_PAD = 10112
RPT = N_PAD // NS
TB = 2000
CW = 8
NBUF = 4


def _seg_sum_kernel(F, with_cnt):
    mesh = plsc.VectorSubcoreMesh(core_axis_name="c", subcore_axis_name="s")
    out_type = [jax.ShapeDtypeStruct((NC, N_PAD, F), jnp.float32)]
    scratch = [
        pltpu.VMEM((R, CHUNK), jnp.int32),
        pltpu.VMEM((R, CHUNK), jnp.int32),
        pltpu.VMEM((NBUF, CHUNK, F), jnp.float32),
        pltpu.VMEM_SHARED((N_PAD, F), jnp.float32),
        pltpu.VMEM_SHARED((N_PAD, F), jnp.float32),
        pltpu.SemaphoreType.DMA((NBUF,)),
        pltpu.SemaphoreType.DMA((NBUF,)),
        pltpu.SemaphoreType.DMA,
    ]
    if with_cnt:
        out_type.append(jax.ShapeDtypeStruct((NC, N_PAD, CW), jnp.float32))
        scratch += [
            pltpu.VMEM((CHUNK, CW), jnp.float32),
            pltpu.VMEM_SHARED((N_PAD, CW), jnp.float32),
        ]

    def body(src_hbm, dst_hbm, y_hbm, zacc_hbm, ones_hbm, zcnt_hbm,
             *rest):
        if with_cnt:
            (out_hbm, cnt_hbm,
             src_v, dst_v, rows_v, y_sh, acc_sh, gsems, ssems, csem,
             ones_v, cnt_sh) = rest
        else:
            (out_hbm, src_v, dst_v, rows_v, y_sh, acc_sh, gsems, ssems,
             csem) = rest
        c = lax.axis_index("c")
        s = lax.axis_index("s")
        wid = s * NC + c
        pltpu.sync_copy(y_hbm.at[pl.ds(s * RPT, RPT)],
                        y_sh.at[pl.ds(s * RPT, RPT)])
        pltpu.sync_copy(zacc_hbm, acc_sh.at[pl.ds(s * RPT, RPT)])
        if with_cnt:
            pltpu.sync_copy(zcnt_hbm, cnt_sh.at[pl.ds(s * RPT, RPT)])
            pltpu.sync_copy(ones_hbm, ones_v)
        pltpu.sync_copy(src_hbm.at[pl.ds(wid * R, R)], src_v)
        pltpu.sync_copy(dst_hbm.at[pl.ds(wid * R, R)], dst_v)
        plsc.subcore_barrier()

        for b in range(NBUF):
            pltpu.async_copy(y_sh.at[src_v.at[b]], rows_v.at[b], gsems.at[b])

        def group(g, carry):
            j0 = g * NBUF
            for b in range(NBUF):
                j = j0 + b
                pltpu.make_async_copy(y_sh.at[src_v.at[j]], rows_v.at[b],
                                      gsems.at[b]).wait()
                pltpu.async_copy(rows_v.at[b], acc_sh.at[dst_v.at[j]],
                                 ssems.at[b], add=True)
                if with_cnt:
                    pltpu.async_copy(ones_v, cnt_sh.at[dst_v.at[j]], csem,
                                     add=True)
                pltpu.make_async_copy(rows_v.at[b], acc_sh.at[dst_v.at[j]],
                                      ssems.at[b]).wait()
                if with_cnt:
                    pltpu.make_async_copy(ones_v, cnt_sh.at[dst_v.at[j]],
                                          csem).wait()
                jn = lax.rem(j + NBUF, R)
                pltpu.async_copy(y_sh.at[src_v.at[jn]], rows_v.at[b],
                                 gsems.at[b])
            return carry

        lax.fori_loop(0, R // NBUF, group, 0)
        for b in range(NBUF):
            pltpu.make_async_copy(y_sh.at[src_v.at[b]], rows_v.at[b],
                                  gsems.at[b]).wait()
        plsc.subcore_barrier()
        pltpu.sync_copy(acc_sh.at[pl.ds(s * RPT, RPT)],
                        out_hbm.at[c, pl.ds(s * RPT, RPT)])
        if with_cnt:
            pltpu.sync_copy(cnt_sh.at[pl.ds(s * RPT, RPT)],
                            cnt_hbm.at[c, pl.ds(s * RPT, RPT)])

    return pl.kernel(body, out_type=out_type if with_cnt else out_type[0],
                     mesh=mesh, scratch_types=scratch,
                     compiler_params=pltpu.CompilerParams(
                         use_tc_tiling_on_sc=False))


_DN = (((1,), (1,)), ((), ()))


def _t1_body(x_ref, wl_ref, wr_ref, b_ref, y_ref, r_ref):
    x = x_ref[...]
    y_ref[...] = lax.dot_general(x, wl_ref[...], _DN,
                                 preferred_element_type=jnp.float32)
    r_ref[...] = lax.dot_general(x, wr_ref[...], _DN,
                                 preferred_element_type=jnp.float32) + b_ref[...]


def _t2_body(acc_ref, cnt_ref, r1_ref, wl_ref, wr_ref, b_ref,
             y2_ref, r2_ref, inv_ref):
    agg = acc_ref[0] + acc_ref[1]
    cnt = cnt_ref[0, :, 0:1] + cnt_ref[1, :, 0:1]
    inv = 1.0 / jnp.maximum(cnt, 1.0)
    h1 = jnp.maximum(agg * inv + r1_ref[...], 0.0)
    y2_ref[...] = lax.dot_general(h1, wl_ref[...], _DN,
                                  preferred_element_type=jnp.float32)
    r2_ref[...] = lax.dot_general(h1, wr_ref[...], _DN,
                                  preferred_element_type=jnp.float32) + b_ref[...]
    inv_ref[...] = jnp.broadcast_to(inv, (TB, H2))


def _t3_body(acc_ref, r2_ref, inv_ref, wlin_ref, blin_ref, out_ref):
    agg = acc_ref[0] + acc_ref[1]
    h2 = jnp.maximum(agg * inv_ref[...] + r2_ref[...], 0.0)
    out_ref[...] = lax.dot_general(h2, wlin_ref[...], _DN,
                                   preferred_element_type=jnp.float32) + blin_ref[...]


def _row_blocked(shape3=False, width=None):
    if shape3:
        return pl.BlockSpec((NC, TB, width), lambda i: (0, i, 0))
    return pl.BlockSpec((TB, width), lambda i: (i, 0))


def _full(shape):
    return pl.BlockSpec(shape, lambda i: tuple(0 for _ in shape))


_t1 = pl.pallas_call(
    _t1_body,
    grid=(N // TB,),
    in_specs=[_row_blocked(width=D), _full((H1, D)), _full((H1, D)),
              _full((1, H1))],
    out_specs=[_row_blocked(width=H1), _row_blocked(width=H1)],
    out_shape=[
        jax.ShapeDtypeStruct((N_PAD, H1), jnp.float32),
        jax.ShapeDtypeStruct((N_PAD, H1), jnp.float32),
    ])
_t2 = pl.pallas_call(
    _t2_body,
    grid=(N // TB,),
    in_specs=[_row_blocked(True, H1), _row_blocked(True, CW),
              _row_blocked(width=H1), _full((H2, H1)), _full((H2, H1)),
              _full((1, H2))],
    out_specs=[_row_blocked(width=H2), _row_blocked(width=H2),
               _row_blocked(width=H2)],
    out_shape=[
        jax.ShapeDtypeStruct((N_PAD, H2), jnp.float32),
        jax.ShapeDtypeStruct((N_PAD, H2), jnp.float32),
        jax.ShapeDtypeStruct((N_PAD, H2), jnp.float32),
    ])
_t3 = pl.pallas_call(
    _t3_body,
    grid=(N // TB,),
    in_specs=[_row_blocked(True, H2), _row_blocked(width=H2),
              _row_blocked(width=H2), _full((OUT, H2)), _full((1, OUT))],
    out_specs=_row_blocked(width=OUT),
    out_shape=jax.ShapeDtypeStruct((N, OUT), jnp.float32))

_s1 = _seg_sum_kernel(H1, with_cnt=True)
_s2 = _seg_sum_kernel(H2, with_cnt=False)


def kernel(x, edge_index, W1l, b1, W1r, W2l, b2, W2r, Wlin, blin):
    pad = E_PAD - E
    trash = N + (jnp.arange(pad, dtype=jnp.int32) % (N_PAD - N))
    src = jnp.concatenate([edge_index[0], jnp.zeros((pad,), jnp.int32)])
    dst = jnp.concatenate([edge_index[1], trash])
    src = src.reshape(NROWS, CHUNK)
    dst = dst.reshape(NROWS, CHUNK)
    zacc1 = jnp.zeros((RPT, H1), jnp.float32)
    zacc2 = jnp.zeros((RPT, H2), jnp.float32)
    zcnt = jnp.zeros((RPT, CW), jnp.float32)
    ones = jnp.ones((CHUNK, CW), jnp.float32)

    y1, r1 = _t1(x, W1l, W1r, b1.reshape(1, H1))
    acc1, cntp = _s1(src, dst, y1, zacc1, ones, zcnt)
    y2, r2, inv = _t2(acc1, cntp, r1, W2l, W2r, b2.reshape(1, H2))
    acc2 = _s2(src, dst, y2, zacc2, ones, zcnt)
    out = _t3(acc2, r2, inv, Wlin, blin.reshape(1, OUT))
    return out


# speedup vs baseline: 1.1033x; 1.0092x over previous
import functools

import jax
import jax.numpy as jnp
from jax import lax
from jax.experimental import pallas as pl
from jax.experimental.pallas import tpu as pltpu
from jax.experimental.pallas import tpu_sc as plsc

N = 10000
E = 320000
D = 128
H1 = 32
H2 = 16
OUT = 2

NC = 2
NS = 16
NW = NC * NS
CHUNK = 128
NROWS = 2560
E_PAD = NROWS * CHUNK
R = NROWS // NW
N_PAD = 10112
RPT = N_PAD // NS
TB = 2000
CW = 8
NBUF = 4


def _seg_sum_kernel(F, with_cnt):
    mesh = plsc.VectorSubcoreMesh(core_axis_name="c", subcore_axis_name="s")
    out_type = [jax.ShapeDtypeStruct((NC, N_PAD, F), jnp.float32)]
    scratch = [
        pltpu.VMEM((R, CHUNK), jnp.int32),
        pltpu.VMEM((R, CHUNK), jnp.int32),
        pltpu.VMEM((NBUF, CHUNK, F), jnp.float32),
        pltpu.VMEM_SHARED((N_PAD, F), jnp.float32),
        pltpu.VMEM_SHARED((N_PAD, F), jnp.float32),
        pltpu.SemaphoreType.DMA((NBUF,)),
        pltpu.SemaphoreType.DMA((NBUF,)),
        pltpu.SemaphoreType.DMA,
    ]
    if with_cnt:
        out_type.append(jax.ShapeDtypeStruct((NC, N_PAD, CW), jnp.float32))
        scratch += [
            pltpu.VMEM((CHUNK, CW), jnp.float32),
            pltpu.VMEM_SHARED((N_PAD, CW), jnp.float32),
        ]

    def body(src_hbm, dst_hbm, y_hbm, zacc_hbm, ones_hbm, zcnt_hbm,
             *rest):
        if with_cnt:
            (out_hbm, cnt_hbm,
             src_v, dst_v, rows_v, y_sh, acc_sh, gsems, ssems, csem,
             ones_v, cnt_sh) = rest
        else:
            (out_hbm, src_v, dst_v, rows_v, y_sh, acc_sh, gsems, ssems,
             csem) = rest
        c = lax.axis_index("c")
        s = lax.axis_index("s")
        wid = s * NC + c
        pltpu.sync_copy(y_hbm.at[pl.ds(s * RPT, RPT)],
                        y_sh.at[pl.ds(s * RPT, RPT)])
        pltpu.sync_copy(zacc_hbm, acc_sh.at[pl.ds(s * RPT, RPT)])
        if with_cnt:
            pltpu.sync_copy(zcnt_hbm, cnt_sh.at[pl.ds(s * RPT, RPT)])
            pltpu.sync_copy(ones_hbm, ones_v)
        pltpu.sync_copy(src_hbm.at[pl.ds(wid * R, R)], src_v)
        pltpu.sync_copy(dst_hbm.at[pl.ds(wid * R, R)], dst_v)
        plsc.subcore_barrier()

        for b in range(NBUF):
            pltpu.async_copy(y_sh.at[src_v.at[b]], rows_v.at[b], gsems.at[b])

        def group(g, carry):
            j0 = g * NBUF
            for b in range(NBUF):
                j = j0 + b
                pltpu.make_async_copy(y_sh.at[src_v.at[j]], rows_v.at[b],
                                      gsems.at[b]).wait()
                pltpu.async_copy(rows_v.at[b], acc_sh.at[dst_v.at[j]],
                                 ssems.at[b], add=True)
                if with_cnt:
                    pltpu.async_copy(ones_v, cnt_sh.at[dst_v.at[j]], csem,
                                     add=True)
                pltpu.make_async_copy(rows_v.at[b], acc_sh.at[dst_v.at[j]],
                                      ssems.at[b]).wait()
                jn = lax.rem(j + NBUF, R)
                pltpu.async_copy(y_sh.at[src_v.at[jn]], rows_v.at[b],
                                 gsems.at[b])
                if with_cnt:
                    pltpu.make_async_copy(ones_v, cnt_sh.at[dst_v.at[j]],
                                          csem).wait()
            return carry

        lax.fori_loop(0, R // NBUF, group, 0)
        for b in range(NBUF):
            pltpu.make_async_copy(y_sh.at[src_v.at[b]], rows_v.at[b],
                                  gsems.at[b]).wait()
        plsc.subcore_barrier()
        pltpu.sync_copy(acc_sh.at[pl.ds(s * RPT, RPT)],
                        out_hbm.at[c, pl.ds(s * RPT, RPT)])
        if with_cnt:
            pltpu.sync_copy(cnt_sh.at[pl.ds(s * RPT, RPT)],
                            cnt_hbm.at[c, pl.ds(s * RPT, RPT)])

    return pl.kernel(body, out_type=out_type if with_cnt else out_type[0],
                     mesh=mesh, scratch_types=scratch,
                     compiler_params=pltpu.CompilerParams(
                         use_tc_tiling_on_sc=False))


_DN = (((1,), (1,)), ((), ()))


def _t1_body(x_ref, wl_ref, wr_ref, b_ref, y_ref, r_ref):
    x = x_ref[...]
    y_ref[...] = lax.dot_general(x, wl_ref[...], _DN,
                                 preferred_element_type=jnp.float32)
    r_ref[...] = lax.dot_general(x, wr_ref[...], _DN,
                                 preferred_element_type=jnp.float32) + b_ref[...]


def _t2_body(acc_ref, cnt_ref, r1_ref, wl_ref, wr_ref, b_ref,
             y2_ref, r2_ref, inv_ref):
    agg = acc_ref[0] + acc_ref[1]
    cnt = cnt_ref[0, :, 0:1] + cnt_ref[1, :, 0:1]
    inv = 1.0 / jnp.maximum(cnt, 1.0)
    h1 = jnp.maximum(agg * inv + r1_ref[...], 0.0)
    y2_ref[...] = lax.dot_general(h1, wl_ref[...], _DN,
                                  preferred_element_type=jnp.float32)
    r2_ref[...] = lax.dot_general(h1, wr_ref[...], _DN,
                                  preferred_element_type=jnp.float32) + b_ref[...]
    inv_ref[...] = jnp.broadcast_to(inv, (TB, H2))


def _t3_body(acc_ref, r2_ref, inv_ref, wlin_ref, blin_ref, out_ref):
    agg = acc_ref[0] + acc_ref[1]
    h2 = jnp.maximum(agg * inv_ref[...] + r2_ref[...], 0.0)
    out_ref[...] = lax.dot_general(h2, wlin_ref[...], _DN,
                                   preferred_element_type=jnp.float32) + blin_ref[...]


def _row_blocked(shape3=False, width=None):
    if shape3:
        return pl.BlockSpec((NC, TB, width), lambda i: (0, i, 0))
    return pl.BlockSpec((TB, width), lambda i: (i, 0))


def _full(shape):
    return pl.BlockSpec(shape, lambda i: tuple(0 for _ in shape))


_t1 = pl.pallas_call(
    _t1_body,
    grid=(N // TB,),
    in_specs=[_row_blocked(width=D), _full((H1, D)), _full((H1, D)),
              _full((1, H1))],
    out_specs=[_row_blocked(width=H1), _row_blocked(width=H1)],
    out_shape=[
        jax.ShapeDtypeStruct((N_PAD, H1), jnp.float32),
        jax.ShapeDtypeStruct((N_PAD, H1), jnp.float32),
    ])
_t2 = pl.pallas_call(
    _t2_body,
    grid=(N // TB,),
    in_specs=[_row_blocked(True, H1), _row_blocked(True, CW),
              _row_blocked(width=H1), _full((H2, H1)), _full((H2, H1)),
              _full((1, H2))],
    out_specs=[_row_blocked(width=H2), _row_blocked(width=H2),
               _row_blocked(width=H2)],
    out_shape=[
        jax.ShapeDtypeStruct((N_PAD, H2), jnp.float32),
        jax.ShapeDtypeStruct((N_PAD, H2), jnp.float32),
        jax.ShapeDtypeStruct((N_PAD, H2), jnp.float32),
    ])
_t3 = pl.pallas_call(
    _t3_body,
    grid=(N // TB,),
    in_specs=[_row_blocked(True, H2), _row_blocked(width=H2),
              _row_blocked(width=H2), _full((OUT, H2)), _full((1, OUT))],
    out_specs=_row_blocked(width=OUT),
    out_shape=jax.ShapeDtypeStruct((N, OUT), jnp.float32))

_s1 = _seg_sum_kernel(H1, with_cnt=True)
_s2 = _seg_sum_kernel(H2, with_cnt=False)


def kernel(x, edge_index, W1l, b1, W1r, W2l, b2, W2r, Wlin, blin):
    pad = E_PAD - E
    trash = N + (jnp.arange(pad, dtype=jnp.int32) % (N_PAD - N))
    src = jnp.concatenate([edge_index[0], jnp.zeros((pad,), jnp.int32)])
    dst = jnp.concatenate([edge_index[1], trash])
    src = src.reshape(NROWS, CHUNK)
    dst = dst.reshape(NROWS, CHUNK)
    zacc1 = jnp.zeros((RPT, H1), jnp.float32)
    zacc2 = jnp.zeros((RPT, H2), jnp.float32)
    zcnt = jnp.zeros((RPT, CW), jnp.float32)
    ones = jnp.ones((CHUNK, CW), jnp.float32)

    y1, r1 = _t1(x, W1l, W1r, b1.reshape(1, H1))
    acc1, cntp = _s1(src, dst, y1, zacc1, ones, zcnt)
    y2, r2, inv = _t2(acc1, cntp, r1, W2l, W2r, b2.reshape(1, H2))
    acc2 = _s2(src, dst, y2, zacc2, ones, zcnt)
    out = _t3(acc2, r2, inv, Wlin, blin.reshape(1, OUT))
    return out
